# baseline (speedup 1.0000x reference)
import functools

import jax
import jax.numpy as jnp
from jax import lax
from jax.experimental import pallas as pl
from jax.experimental.pallas import tpu as pltpu
from jax.experimental.pallas import tpu_sc as plsc

NUM_FIXED = 900000
EMBED_DIM = 32
NC = 2
NS = 16
L = 16
NW = NC * NS

CHUNK = 640
GBLK = CHUNK // 128
BLK = 128


@jax.jit
def _lookup(idx, fixed_weights, trainable_weight):
    b_total = idx.shape[0]
    assert b_total % NW == 0
    bpw = b_total // NW
    assert bpw % CHUNK == 0
    nchunk = bpw // CHUNK
    stage_cap = bpw + BLK
    trash_slot = stage_cap - L
    trash = b_total

    mesh = plsc.VectorSubcoreMesh(
        core_axis_name="c", subcore_axis_name="s", num_cores=NC, num_subcores=NS
    )

    @functools.partial(
        pl.kernel,
        out_type=jax.ShapeDtypeStruct((b_total + 8, EMBED_DIM), jnp.float32),
        mesh=mesh,
        scratch_types=[
            pltpu.VMEM((CHUNK,), jnp.int32),
            pltpu.VMEM((GBLK, BLK), jnp.int32),
            pltpu.VMEM((CHUNK, EMBED_DIM), jnp.float32),
            pltpu.VMEM((stage_cap,), jnp.int32),
            pltpu.VMEM((stage_cap,), jnp.int32),
            pltpu.VMEM((BLK,), jnp.int32),
            pltpu.VMEM((BLK,), jnp.int32),
            pltpu.VMEM((BLK, EMBED_DIM), jnp.float32),
            pltpu.SemaphoreType.DMA,
            pltpu.SemaphoreType.DMA,
        ],
        compiler_params=pltpu.CompilerParams(
            needs_layout_passes=False, use_tc_tiling_on_sc=False),
    )
    def k(idx_hbm, fixed_hbm, train_hbm, out_hbm,
          idx_v, clamp_v, rows_v, tidx_st, pos_st, tblk, pblk, rows_t,
          sem, sem2):
        wid = lax.axis_index("s") * NC + lax.axis_index("c")
        base = wid * bpw

        def fill(i, _):
            tidx_st[pl.ds(i * L, L)] = jnp.zeros((L,), jnp.int32)
            pos_st[pl.ds(i * L, L)] = jnp.full((L,), trash, jnp.int32)
            return 0
        lax.fori_loop(0, stage_cap // L, fill, 0)

        iota = lax.iota(jnp.int32, L)

        def chunk_body(c, n_t):
            cbase = base + c * CHUNK
            pltpu.sync_copy(idx_hbm.at[pl.ds(cbase, CHUNK)], idx_v)

            for j in range(GBLK):
                def grp(g, n_t):
                    off = j * BLK + g * L
                    v = idx_v[pl.ds(off, L)]
                    m = v >= NUM_FIXED
                    clamp_v[j, pl.ds(g * L, L)] = jnp.where(m, 0, v)
                    mi = m.astype(jnp.int32)
                    dest = jnp.where(
                        m, n_t + plsc.cumsum(mi) - 1, trash_slot + iota)
                    plsc.store_scatter(tidx_st, [dest], v - NUM_FIXED)
                    plsc.store_scatter(pos_st, [dest], cbase + off + iota)
                    return n_t + jnp.sum(mi)
                n_t = lax.fori_loop(0, BLK // L, grp, n_t)

            handles = [
                pltpu.async_copy(
                    fixed_hbm.at[clamp_v.at[j]],
                    rows_v.at[pl.ds(j * BLK, BLK)],
                    sem,
                )
                for j in range(GBLK)
            ]
            for h in handles:
                h.wait()
            pltpu.sync_copy(rows_v, out_hbm.at[pl.ds(cbase, CHUNK)])
            return n_t

        n_t = lax.fori_loop(0, nchunk, chunk_body, jnp.int32(0))

        nblk = (n_t + (BLK - 1)) // BLK

        def blk_body(b, _):
            for g in range(BLK // L):
                tblk[pl.ds(g * L, L)] = tidx_st[pl.ds(b * BLK + g * L, L)]
                pblk[pl.ds(g * L, L)] = pos_st[pl.ds(b * BLK + g * L, L)]
            pltpu.async_copy(train_hbm.at[tblk], rows_t, sem).wait()
            pltpu.async_copy(rows_t, out_hbm.at[pblk], sem2).wait()
            return 0
        lax.fori_loop(0, nblk, blk_body, 0)

    return k(idx, fixed_weights, trainable_weight)


def kernel(inp, fixed_weights, trainable_weight):
    nb, nh = inp.shape
    idx = inp.reshape(-1).astype(jnp.int32)
    out = _lookup(idx, fixed_weights, trainable_weight)
    return out[: nb * nh].reshape(nb, nh, EMBED_DIM)
